# baseline (speedup 1.0000x reference)
import functools

import jax
import jax.numpy as jnp
from jax import lax
from jax.experimental import pallas as pl
from jax.experimental.pallas import tpu as pltpu
import jax.experimental.pallas.tpu_sc as plsc

N = 10000
E = 320000
D = 128
H = 256
G = 64
A = 8
O = 1

EC = E + N
NT = 16
NC = 2
EPT = 20736
ECP = NT * EPT
SB = 2304
NSB = EPT // SB
K = 128
NKB = SB // K
NP = 10240
RPT = NP // NT
NBUF = 3
NQ = 4
DQ = H // NQ

_f32 = jnp.float32


def _sc_edge_body(h4, asrc_h, adst_h, srcp, dstp, cvec_h,
                  sums_o, den_o, den_pub,
                  asrc_v, adst_v, den_v, ex_v, s_sb, d_sb, gidx_v, didx_v,
                  rows_v, part_v, acc_v, cvec_v, sem_g, sem_s,
                  out_acc):
    c = lax.axis_index("c")
    s = lax.axis_index("s")
    tb = s * EPT
    row0 = s * RPT

    zero16 = jnp.zeros((16,), _f32)

    def _zden(i, _):
        den_v[pl.ds(i * 16, 16)] = zero16
        return 0
    lax.fori_loop(0, NP // 16, _zden, 0)

    def _zzr(i, _):
        for j in range(DQ // 16):
            rows_v[0][i, pl.ds(j * 16, 16)] = zero16
        return 0
    lax.fori_loop(0, K, _zzr, 0)

    for p in range(RPT // K):
        pltpu.sync_copy(rows_v[0], out_acc.at[pl.ds(row0 + p * K, K)])
    _zr = RPT - (RPT // K) * K
    if _zr:
        pltpu.sync_copy(rows_v[0].at[pl.ds(0, _zr)],
                        out_acc.at[pl.ds(row0 + (RPT // K) * K, _zr)])

    pltpu.sync_copy(asrc_h, asrc_v)
    pltpu.sync_copy(adst_h, adst_v)
    pltpu.sync_copy(cvec_h, cvec_v)
    cval = cvec_v[...][0]

    _ns_p1 = jax.named_scope("sc_phase1")
    _ns_p1.__enter__()

    def _p1_sb(sb, _):
        base = tb + sb * SB
        pltpu.sync_copy(srcp.at[pl.ds(base, SB)], s_sb)
        pltpu.sync_copy(dstp.at[pl.ds(base, SB)], d_sb)

        def _p1(j, _):
            sv = s_sb[pl.ds(j * 16, 16)]
            dv = d_sb[pl.ds(j * 16, 16)]
            av = plsc.load_gather(asrc_v, [sv])
            bv = plsc.load_gather(adst_v, [dv])
            t = av + bv
            e = jnp.where(t >= 0.0, t, t * 0.2)
            ex = jnp.exp(e - cval)
            gid = base + j * 16 + lax.iota(jnp.int32, 16)
            ex = jnp.where(gid < EC, ex, 0.0)
            ex_v[pl.ds(sb * SB + j * 16, 16)] = ex
            plsc.addupdate_scatter(den_v, [dv], ex)
            return 0
        lax.fori_loop(0, SB // 16, _p1, 0)
        return 0
    lax.fori_loop(0, NSB, _p1_sb, 0)
    _ns_p1.__exit__(None, None, None)

    _ns_dm = jax.named_scope("sc_denmerge")
    _ns_dm.__enter__()
    pltpu.sync_copy(den_v, den_pub.at[c, s])
    plsc.subcore_barrier()

    pltpu.sync_copy(den_pub.at[c, 0, pl.ds(row0, RPT)], acc_v)
    for t in range(1, NT):
        pltpu.sync_copy(den_pub.at[c, t, pl.ds(row0, RPT)], part_v)

        def _acc(i, _):
            acc_v[pl.ds(i * 16, 16)] = (acc_v[pl.ds(i * 16, 16)]
                                        + part_v[pl.ds(i * 16, 16)])
            return 0
        lax.fori_loop(0, RPT // 16, _acc, 0)
    pltpu.sync_copy(acc_v, den_o.at[c, pl.ds(row0, RPT)])
    _ns_dm.__exit__(None, None, None)

    for p in range(2):
        _ns_p2 = jax.named_scope(f"sc_phase2_{p}")
        _ns_p2.__enter__()
        qbase = (2 * c + p) * N

        if p > 0:
            def _zzr2(i, _):
                for j in range(DQ // 16):
                    rows_v[0][i, pl.ds(j * 16, 16)] = zero16
                return 0
            lax.fori_loop(0, K, _zzr2, 0)
            for z in range(RPT // K):
                pltpu.sync_copy(rows_v[0], out_acc.at[pl.ds(row0 + z * K, K)])
            _zr2 = RPT - (RPT // K) * K
            if _zr2:
                pltpu.sync_copy(rows_v[0].at[pl.ds(0, _zr2)],
                                out_acc.at[pl.ds(row0 + (RPT // K) * K, _zr2)])
            plsc.subcore_barrier()

        def _p2_sb(sb, _, qbase=qbase):
            base = tb + sb * SB
            pltpu.sync_copy(srcp.at[pl.ds(base, SB)], s_sb)
            pltpu.sync_copy(dstp.at[pl.ds(base, SB)], d_sb)

            def _p2_grp(grp, _):
                gds = []
                for t in range(NBUF):
                    koff = (grp * NBUF + t) * K
                    for j in range(K // 16):
                        gidx_v[t][pl.ds(j * 16, 16)] = (
                            s_sb[pl.ds(koff + j * 16, 16)] + qbase)
                        didx_v[t][pl.ds(j * 16, 16)] = (
                            d_sb[pl.ds(koff + j * 16, 16)])
                    gds.append(pltpu.async_copy(h4.at[gidx_v[t]], rows_v[t],
                                                sem_g[t]))
                sds = []
                for t in range(NBUF):
                    gds[t].wait()
                    exbase = sb * SB + (grp * NBUF + t) * K

                    def _row_grp(g, _, t=t, exbase=exbase):
                        bv = jnp.full((16,), exbase + g * 16, jnp.int32)
                        for l in range(16):
                            exb = plsc.load_gather(ex_v, [bv + l])
                            i = g * 16 + l
                            for j in range(DQ // 16):
                                rows_v[t][i, pl.ds(j * 16, 16)] = (
                                    rows_v[t][i, pl.ds(j * 16, 16)] * exb)
                        return 0
                    lax.fori_loop(0, K // 16, _row_grp, 0)
                    sds.append(pltpu.async_copy(rows_v[t],
                                                out_acc.at[didx_v[t]],
                                                sem_s[t], add=True))
                for t in range(NBUF):
                    sds[t].wait()
                return 0
            lax.fori_loop(0, NKB // NBUF, _p2_grp, 0)
            return 0
        lax.fori_loop(0, NSB, _p2_sb, 0)

        plsc.subcore_barrier()

        pltpu.sync_copy(out_acc.at[pl.ds(row0, RPT)],
                        sums_o.at[pl.ds((2 * c + p) * NP + row0, RPT)])
        _ns_p2.__exit__(None, None, None)


@functools.partial(jax.jit, static_argnames=())
def _sc_edge(h4, asrc, adst, srcp, dstp, cvec):
    mesh = plsc.VectorSubcoreMesh(core_axis_name="c", subcore_axis_name="s")
    return pl.kernel(
        _sc_edge_body,
        out_type=[
            jax.ShapeDtypeStruct((NQ * NP, DQ), _f32),
            jax.ShapeDtypeStruct((NC, NP), _f32),
            jax.ShapeDtypeStruct((NC, NT, NP), _f32),
        ],
        mesh=mesh,
        compiler_params=pltpu.CompilerParams(needs_layout_passes=False,
                                             use_tc_tiling_on_sc=False),
        scratch_types=[
            pltpu.VMEM((N,), _f32),
            pltpu.VMEM((N,), _f32),
            pltpu.VMEM((NP,), _f32),
            pltpu.VMEM((EPT,), _f32),
            pltpu.VMEM((SB,), jnp.int32),
            pltpu.VMEM((SB,), jnp.int32),
            [pltpu.VMEM((K,), jnp.int32) for _ in range(NBUF)],
            [pltpu.VMEM((K,), jnp.int32) for _ in range(NBUF)],
            [pltpu.VMEM((K, DQ), _f32) for _ in range(NBUF)],
            pltpu.VMEM((RPT,), _f32),
            pltpu.VMEM((RPT,), _f32),
            pltpu.VMEM((16,), _f32),
            [pltpu.SemaphoreType.DMA for _ in range(NBUF)],
            [pltpu.SemaphoreType.DMA for _ in range(NBUF)],
            pltpu.VMEM_SHARED((NP, DQ), _f32),
        ],
    )(h4, asrc, adst, srcp, dstp, cvec)[:2]


BN = 2000
NBL = N // BN


def _leaky(t):
    return jnp.where(t >= 0.0, t, t * 0.2)


def _attn_block(i, hn, asn_ref, adn_ref, asrc_ref, adst_ref, c_ref, mx_ref):
    asrc = jnp.dot(hn, asn_ref[...], preferred_element_type=_f32, precision=lax.Precision.HIGHEST)
    adst = jnp.dot(hn, adn_ref[...], preferred_element_type=_f32, precision=lax.Precision.HIGHEST)
    asrc_ref[...] = asrc
    adst_ref[...] = adst
    bs = jnp.max(asrc)
    bd = jnp.max(adst)

    @pl.when(i == 0)
    def _():
        mx_ref[0] = bs
        mx_ref[1] = bd

    @pl.when(i > 0)
    def _():
        mx_ref[0] = jnp.maximum(mx_ref[0], bs)
        mx_ref[1] = jnp.maximum(mx_ref[1], bd)

    @pl.when(i == NBL - 1)
    def _():
        c_ref[...] = jnp.full((1, 16), _leaky(mx_ref[0] + mx_ref[1]), _f32)


def _pre_body(x_ref, w_ref, asn_ref, adn_ref,
              h4_ref, asrc_ref, adst_ref, c_ref, mx_ref):
    i = pl.program_id(0)
    hn = jnp.dot(x_ref[...], w_ref[...], preferred_element_type=_f32, precision=lax.Precision.HIGHEST)
    for q in range(NQ):
        h4_ref[q] = hn[:, q * DQ:(q + 1) * DQ]
    _attn_block(i, hn, asn_ref, adn_ref, asrc_ref, adst_ref, c_ref, mx_ref)


def _norm_block(sums_ref, denc, b, g, be):
    u = jnp.concatenate([sums_ref[q] for q in range(NQ)], axis=1)
    v = u / (denc + 1e-16) + b
    mu = jnp.mean(v, axis=1, keepdims=True)
    var = jnp.mean((v - mu) ** 2, axis=1, keepdims=True)
    ln = (v - mu) / jnp.sqrt(var + 1e-5) * g + be
    return jnp.maximum(ln, 0.0)


def _mid_body(sums_ref, denc_ref, b_ref, g_ref, be_ref, w_ref, asn_ref, adn_ref,
              h4_ref, asrc_ref, adst_ref, c_ref, mx_ref):
    i = pl.program_id(0)
    r = _norm_block(sums_ref, denc_ref[...], b_ref[...], g_ref[...], be_ref[...])
    hn = jnp.dot(r, w_ref[...], preferred_element_type=_f32, precision=lax.Precision.HIGHEST)
    for q in range(NQ):
        h4_ref[q] = hn[:, q * DQ:(q + 1) * DQ]
    _attn_block(i, hn, asn_ref, adn_ref, asrc_ref, adst_ref, c_ref, mx_ref)


def _post_body(sums_ref, denc_ref, b_ref, g_ref, be_ref, batch_ref, ga_ref,
               f1a_ref, f1b_ref, f1bias_ref, f2w_ref, f2b_ref, out_ref,
               pool_ref, cnt_ref):
    i = pl.program_id(0)
    h = _norm_block(sums_ref, denc_ref[...], b_ref[...], g_ref[...],
                    be_ref[...])
    iota_g = lax.broadcasted_iota(jnp.int32, (G, BN), 0)
    onehot = (batch_ref[0] == iota_g).astype(_f32)
    cnt = jnp.sum(onehot, axis=1, keepdims=True)
    pooled = jnp.dot(onehot, h, preferred_element_type=_f32, precision=lax.Precision.HIGHEST)

    @pl.when(i == 0)
    def _():
        pool_ref[...] = pooled
        cnt_ref[...] = cnt

    @pl.when(i > 0)
    def _():
        pool_ref[...] = pool_ref[...] + pooled
        cnt_ref[...] = cnt_ref[...] + cnt

    @pl.when(i == NBL - 1)
    def _():
        pooled_m = pool_ref[...] / jnp.maximum(cnt_ref[...], 1.0)
        z = (pooled_m @ f1a_ref[...] + ga_ref[...] @ f1b_ref[...]
             + f1bias_ref[...])
        z = jnp.maximum(z, 0.0)
        out_ref[...] = z @ f2w_ref[...] + f2b_ref[...]


_spec_whole = lambda shape: pl.BlockSpec(shape, lambda i: (0,) * len(shape))
_spec_rows = pl.BlockSpec((BN, 1), lambda i: (i, 0))
_spec_h4 = pl.BlockSpec((NQ, BN, DQ), lambda i: (0, i, 0))

_attn_outs_shape = [
    jax.ShapeDtypeStruct((NQ, N, DQ), _f32),
    jax.ShapeDtypeStruct((N, 1), _f32),
    jax.ShapeDtypeStruct((N, 1), _f32),
    jax.ShapeDtypeStruct((1, 16), _f32),
]
_attn_outs_spec = [_spec_h4, _spec_rows, _spec_rows, _spec_whole((1, 16))]


def _pre_call(x, W, a_s, a_d):
    return pl.pallas_call(
        _pre_body,
        grid=(NBL,),
        in_specs=[pl.BlockSpec((BN, D), lambda i: (i, 0)),
                  _spec_whole((D, H)), _spec_whole((H, 1)), _spec_whole((H, 1))],
        out_specs=_attn_outs_spec,
        out_shape=_attn_outs_shape,
        scratch_shapes=[pltpu.SMEM((2,), _f32)],
    )(x, W, a_s.reshape(H, 1), a_d.reshape(H, 1))


def _mid_call(sums4, denc, b, g, be, W, a_s, a_d):
    return pl.pallas_call(
        _mid_body,
        grid=(NBL,),
        in_specs=[pl.BlockSpec((NQ, BN, DQ), lambda i: (0, i, 0)),
                  _spec_rows,
                  _spec_whole((1, H)), _spec_whole((1, H)), _spec_whole((1, H)),
                  _spec_whole((H, H)), _spec_whole((H, 1)), _spec_whole((H, 1))],
        out_specs=_attn_outs_spec,
        out_shape=_attn_outs_shape,
        scratch_shapes=[pltpu.SMEM((2,), _f32)],
    )(sums4, denc, b.reshape(1, H), g.reshape(1, H), be.reshape(1, H),
      W, a_s.reshape(H, 1), a_d.reshape(H, 1))


def _post_call(sums4, denc, b, g, be, batch, graph_attr, fc1W, fc1b, fc2W, fc2b):
    return pl.pallas_call(
        _post_body,
        grid=(NBL,),
        in_specs=[pl.BlockSpec((NQ, BN, DQ), lambda i: (0, i, 0)),
                  _spec_rows,
                  _spec_whole((1, H)), _spec_whole((1, H)), _spec_whole((1, H)),
                  pl.BlockSpec((1, 1, BN), lambda i: (i, 0, 0)),
                  _spec_whole((G, A)),
                  _spec_whole((H, H)), _spec_whole((A, H)),
                  _spec_whole((1, H)), _spec_whole((H, O)),
                  _spec_whole((1, O))],
        out_specs=_spec_whole((G, O)),
        out_shape=jax.ShapeDtypeStruct((G, O), _f32),
        scratch_shapes=[pltpu.VMEM((G, H), _f32), pltpu.VMEM((G, 1), _f32)],
    )(sums4, denc, b.reshape(1, H), g.reshape(1, H), be.reshape(1, H),
      batch.reshape(NBL, 1, BN), graph_attr.reshape(G, A),
      fc1W[:H], fc1W[H:], fc1b.reshape(1, H), fc2W, fc2b.reshape(1, O))


def kernel(x, edge_index, batch, graph_attr, W1, a1s, a1d, b1, g1, be1, W2, a2s, a2d, b2, g2, be2, W3, a3s, a3d, b3, g3, be3, fc1W, fc1b, fc2W, fc2b):
    loop = jnp.arange(N, dtype=jnp.int32)
    pad = jnp.zeros((ECP - EC,), jnp.int32)
    srcp = jnp.concatenate([edge_index[0].astype(jnp.int32), loop, pad])
    dstp = jnp.concatenate([edge_index[1].astype(jnp.int32), loop, pad])

    h4, asrc, adst, cv = _pre_call(x, W1, a1s, a1d)
    sums, den = _sc_edge(h4.reshape(NQ * N, DQ), asrc.reshape(N),
                         adst.reshape(N), srcp, dstp, cv.reshape(16))

    h4, asrc, adst, cv = _mid_call(sums.reshape(NQ, NP, DQ),
                                   den[0, :N].reshape(N, 1), b1, g1, be1,
                                   W2, a2s, a2d)
    sums, den = _sc_edge(h4.reshape(NQ * N, DQ), asrc.reshape(N),
                         adst.reshape(N), srcp, dstp, cv.reshape(16))

    h4, asrc, adst, cv = _mid_call(sums.reshape(NQ, NP, DQ),
                                   den[0, :N].reshape(N, 1), b2, g2, be2,
                                   W3, a3s, a3d)
    sums, den = _sc_edge(h4.reshape(NQ * N, DQ), asrc.reshape(N),
                         adst.reshape(N), srcp, dstp, cv.reshape(16))

    return _post_call(sums.reshape(NQ, NP, DQ), den[0, :N].reshape(N, 1),
                      b3, g3, be3, batch, graph_attr, fc1W, fc1b, fc2W, fc2b)


# speedup vs baseline: 1.0593x; 1.0593x over previous
import functools

import jax
import jax.numpy as jnp
from jax import lax
from jax.experimental import pallas as pl
from jax.experimental.pallas import tpu as pltpu
import jax.experimental.pallas.tpu_sc as plsc

N = 10000
E = 320000
D = 128
H = 256
G = 64
A = 8
O = 1

EC = E + N
NT = 16
NC = 2
EPT = 20736
ECP = NT * EPT
SB = 2304
NSB = EPT // SB
K = 128
NKB = SB // K
NP = 10240
RPT = NP // NT
NBUF = 3
NQ = 4
DQ = H // NQ

_f32 = jnp.float32


def _sc_edge_body(h4, asrc_h, adst_h, srcp, dstp, cvec_h,
                  sums_o, den_o, den_pub,
                  asrc_v, adst_v, den_v, ex_v, s_sb, d_sb, gidx_v, didx_v,
                  rows_v, part_v, acc_v, cvec_v, sem_g, sem_s,
                  out_acc):
    c = lax.axis_index("c")
    s = lax.axis_index("s")
    tb = s * EPT
    row0 = s * RPT

    zero16 = jnp.zeros((16,), _f32)

    def _zden(i, _):
        den_v[pl.ds(i * 16, 16)] = zero16
        return 0
    lax.fori_loop(0, NP // 16, _zden, 0)

    def _zzr(i, _):
        for j in range(DQ // 16):
            rows_v[0][i, pl.ds(j * 16, 16)] = zero16
        return 0
    lax.fori_loop(0, K, _zzr, 0)

    for p in range(RPT // K):
        pltpu.sync_copy(rows_v[0], out_acc.at[pl.ds(row0 + p * K, K)])
    _zr = RPT - (RPT // K) * K
    if _zr:
        pltpu.sync_copy(rows_v[0].at[pl.ds(0, _zr)],
                        out_acc.at[pl.ds(row0 + (RPT // K) * K, _zr)])

    pltpu.sync_copy(asrc_h, asrc_v)
    pltpu.sync_copy(adst_h, adst_v)
    pltpu.sync_copy(cvec_h, cvec_v)
    cval = cvec_v[...][0]

    _ns_p1 = jax.named_scope("sc_phase1")
    _ns_p1.__enter__()

    def _p1_sb(sb, _):
        base = tb + sb * SB
        pltpu.sync_copy(srcp.at[pl.ds(base, SB)], s_sb)
        pltpu.sync_copy(dstp.at[pl.ds(base, SB)], d_sb)

        def _p1(j, _):
            sv = s_sb[pl.ds(j * 16, 16)]
            dv = d_sb[pl.ds(j * 16, 16)]
            av = plsc.load_gather(asrc_v, [sv])
            bv = plsc.load_gather(adst_v, [dv])
            t = av + bv
            e = jnp.where(t >= 0.0, t, t * 0.2)
            ex = jnp.exp(e - cval)
            gid = base + j * 16 + lax.iota(jnp.int32, 16)
            ex = jnp.where(gid < EC, ex, 0.0)
            ex_v[pl.ds(sb * SB + j * 16, 16)] = ex
            plsc.addupdate_scatter(den_v, [dv], ex)
            return 0
        lax.fori_loop(0, SB // 16, _p1, 0)
        return 0
    lax.fori_loop(0, NSB, _p1_sb, 0)
    _ns_p1.__exit__(None, None, None)

    _ns_dm = jax.named_scope("sc_denmerge")
    _ns_dm.__enter__()
    pltpu.sync_copy(den_v, den_pub.at[c, s])
    plsc.subcore_barrier()

    pltpu.sync_copy(den_pub.at[c, 0, pl.ds(row0, RPT)], acc_v)
    for t in range(1, NT):
        pltpu.sync_copy(den_pub.at[c, t, pl.ds(row0, RPT)], part_v)

        def _acc(i, _):
            acc_v[pl.ds(i * 16, 16)] = (acc_v[pl.ds(i * 16, 16)]
                                        + part_v[pl.ds(i * 16, 16)])
            return 0
        lax.fori_loop(0, RPT // 16, _acc, 0)
    pltpu.sync_copy(acc_v, den_o.at[c, pl.ds(row0, RPT)])
    _ns_dm.__exit__(None, None, None)

    for p in range(2):
        _ns_p2 = jax.named_scope(f"sc_phase2_{p}")
        _ns_p2.__enter__()
        qbase = (2 * c + p) * N

        if p > 0:
            def _zzr2(i, _):
                for j in range(DQ // 16):
                    rows_v[0][i, pl.ds(j * 16, 16)] = zero16
                return 0
            lax.fori_loop(0, K, _zzr2, 0)
            for z in range(RPT // K):
                pltpu.sync_copy(rows_v[0], out_acc.at[pl.ds(row0 + z * K, K)])
            _zr2 = RPT - (RPT // K) * K
            if _zr2:
                pltpu.sync_copy(rows_v[0].at[pl.ds(0, _zr2)],
                                out_acc.at[pl.ds(row0 + (RPT // K) * K, _zr2)])
            plsc.subcore_barrier()

        def _p2_sb(sb, _, qbase=qbase):
            base = tb + sb * SB
            pltpu.sync_copy(srcp.at[pl.ds(base, SB)], s_sb)
            pltpu.sync_copy(dstp.at[pl.ds(base, SB)], d_sb)

            def _p2_grp(grp, _):
                gg = sb * (NKB // NBUF) + grp
                gds = []
                for t in range(NBUF):
                    @pl.when(gg > 0)
                    def _(t=t):
                        pltpu.make_async_copy(
                            rows_v[t], out_acc.at[didx_v[t]], sem_s[t]).wait()
                    koff = (grp * NBUF + t) * K
                    for j in range(K // 16):
                        gidx_v[t][pl.ds(j * 16, 16)] = (
                            s_sb[pl.ds(koff + j * 16, 16)] + qbase)
                        didx_v[t][pl.ds(j * 16, 16)] = (
                            d_sb[pl.ds(koff + j * 16, 16)])
                    gds.append(pltpu.async_copy(h4.at[gidx_v[t]], rows_v[t],
                                                sem_g[t]))
                for t in range(NBUF):
                    gds[t].wait()
                    exbase = sb * SB + (grp * NBUF + t) * K

                    def _row_grp(g, _, t=t, exbase=exbase):
                        bv = jnp.full((16,), exbase + g * 16, jnp.int32)
                        for l in range(16):
                            exb = plsc.load_gather(ex_v, [bv + l])
                            i = g * 16 + l
                            for j in range(DQ // 16):
                                rows_v[t][i, pl.ds(j * 16, 16)] = (
                                    rows_v[t][i, pl.ds(j * 16, 16)] * exb)
                        return 0
                    lax.fori_loop(0, K // 16, _row_grp, 0)
                    pltpu.async_copy(rows_v[t], out_acc.at[didx_v[t]],
                                     sem_s[t], add=True)
                return 0
            lax.fori_loop(0, NKB // NBUF, _p2_grp, 0)
            return 0
        lax.fori_loop(0, NSB, _p2_sb, 0)

        for t in range(NBUF):
            pltpu.make_async_copy(rows_v[t], out_acc.at[didx_v[t]],
                                  sem_s[t]).wait()

        plsc.subcore_barrier()

        pltpu.sync_copy(out_acc.at[pl.ds(row0, RPT)],
                        sums_o.at[pl.ds((2 * c + p) * NP + row0, RPT)])
        _ns_p2.__exit__(None, None, None)


@functools.partial(jax.jit, static_argnames=())
def _sc_edge(h4, asrc, adst, srcp, dstp, cvec):
    mesh = plsc.VectorSubcoreMesh(core_axis_name="c", subcore_axis_name="s")
    return pl.kernel(
        _sc_edge_body,
        out_type=[
            jax.ShapeDtypeStruct((NQ * NP, DQ), _f32),
            jax.ShapeDtypeStruct((NC, NP), _f32),
            jax.ShapeDtypeStruct((NC, NT, NP), _f32),
        ],
        mesh=mesh,
        compiler_params=pltpu.CompilerParams(needs_layout_passes=False,
                                             use_tc_tiling_on_sc=False),
        scratch_types=[
            pltpu.VMEM((N,), _f32),
            pltpu.VMEM((N,), _f32),
            pltpu.VMEM((NP,), _f32),
            pltpu.VMEM((EPT,), _f32),
            pltpu.VMEM((SB,), jnp.int32),
            pltpu.VMEM((SB,), jnp.int32),
            [pltpu.VMEM((K,), jnp.int32) for _ in range(NBUF)],
            [pltpu.VMEM((K,), jnp.int32) for _ in range(NBUF)],
            [pltpu.VMEM((K, DQ), _f32) for _ in range(NBUF)],
            pltpu.VMEM((RPT,), _f32),
            pltpu.VMEM((RPT,), _f32),
            pltpu.VMEM((16,), _f32),
            [pltpu.SemaphoreType.DMA for _ in range(NBUF)],
            [pltpu.SemaphoreType.DMA for _ in range(NBUF)],
            pltpu.VMEM_SHARED((NP, DQ), _f32),
        ],
    )(h4, asrc, adst, srcp, dstp, cvec)[:2]


BN = 2000
NBL = N // BN


def _leaky(t):
    return jnp.where(t >= 0.0, t, t * 0.2)


def _attn_block(i, hn, asn_ref, adn_ref, asrc_ref, adst_ref, c_ref, mx_ref):
    asrc = jnp.dot(hn, asn_ref[...], preferred_element_type=_f32, precision=lax.Precision.HIGHEST)
    adst = jnp.dot(hn, adn_ref[...], preferred_element_type=_f32, precision=lax.Precision.HIGHEST)
    asrc_ref[...] = asrc
    adst_ref[...] = adst
    bs = jnp.max(asrc)
    bd = jnp.max(adst)

    @pl.when(i == 0)
    def _():
        mx_ref[0] = bs
        mx_ref[1] = bd

    @pl.when(i > 0)
    def _():
        mx_ref[0] = jnp.maximum(mx_ref[0], bs)
        mx_ref[1] = jnp.maximum(mx_ref[1], bd)

    @pl.when(i == NBL - 1)
    def _():
        c_ref[...] = jnp.full((1, 16), _leaky(mx_ref[0] + mx_ref[1]), _f32)


def _pre_body(x_ref, w_ref, asn_ref, adn_ref,
              h4_ref, asrc_ref, adst_ref, c_ref, mx_ref):
    i = pl.program_id(0)
    hn = jnp.dot(x_ref[...], w_ref[...], preferred_element_type=_f32, precision=lax.Precision.HIGHEST)
    for q in range(NQ):
        h4_ref[q] = hn[:, q * DQ:(q + 1) * DQ]
    _attn_block(i, hn, asn_ref, adn_ref, asrc_ref, adst_ref, c_ref, mx_ref)


def _norm_block(sums_ref, denc, b, g, be):
    u = jnp.concatenate([sums_ref[q] for q in range(NQ)], axis=1)
    v = u / (denc + 1e-16) + b
    mu = jnp.mean(v, axis=1, keepdims=True)
    var = jnp.mean((v - mu) ** 2, axis=1, keepdims=True)
    ln = (v - mu) / jnp.sqrt(var + 1e-5) * g + be
    return jnp.maximum(ln, 0.0)


def _mid_body(sums_ref, denc_ref, b_ref, g_ref, be_ref, w_ref, asn_ref, adn_ref,
              h4_ref, asrc_ref, adst_ref, c_ref, mx_ref):
    i = pl.program_id(0)
    r = _norm_block(sums_ref, denc_ref[...], b_ref[...], g_ref[...], be_ref[...])
    hn = jnp.dot(r, w_ref[...], preferred_element_type=_f32, precision=lax.Precision.HIGHEST)
    for q in range(NQ):
        h4_ref[q] = hn[:, q * DQ:(q + 1) * DQ]
    _attn_block(i, hn, asn_ref, adn_ref, asrc_ref, adst_ref, c_ref, mx_ref)


def _post_body(sums_ref, denc_ref, b_ref, g_ref, be_ref, batch_ref, ga_ref,
               f1a_ref, f1b_ref, f1bias_ref, f2w_ref, f2b_ref, out_ref,
               pool_ref, cnt_ref):
    i = pl.program_id(0)
    h = _norm_block(sums_ref, denc_ref[...], b_ref[...], g_ref[...],
                    be_ref[...])
    iota_g = lax.broadcasted_iota(jnp.int32, (G, BN), 0)
    onehot = (batch_ref[0] == iota_g).astype(_f32)
    cnt = jnp.sum(onehot, axis=1, keepdims=True)
    pooled = jnp.dot(onehot, h, preferred_element_type=_f32, precision=lax.Precision.HIGHEST)

    @pl.when(i == 0)
    def _():
        pool_ref[...] = pooled
        cnt_ref[...] = cnt

    @pl.when(i > 0)
    def _():
        pool_ref[...] = pool_ref[...] + pooled
        cnt_ref[...] = cnt_ref[...] + cnt

    @pl.when(i == NBL - 1)
    def _():
        pooled_m = pool_ref[...] / jnp.maximum(cnt_ref[...], 1.0)
        _hp = functools.partial(jnp.dot, preferred_element_type=_f32,
                                precision=lax.Precision.HIGHEST)
        z = (_hp(pooled_m, f1a_ref[...]) + _hp(ga_ref[...], f1b_ref[...])
             + f1bias_ref[...])
        z = jnp.maximum(z, 0.0)
        out_ref[...] = _hp(z, f2w_ref[...]) + f2b_ref[...]


_spec_whole = lambda shape: pl.BlockSpec(shape, lambda i: (0,) * len(shape))
_spec_rows = pl.BlockSpec((BN, 1), lambda i: (i, 0))
_spec_h4 = pl.BlockSpec((NQ, BN, DQ), lambda i: (0, i, 0))

_attn_outs_shape = [
    jax.ShapeDtypeStruct((NQ, N, DQ), _f32),
    jax.ShapeDtypeStruct((N, 1), _f32),
    jax.ShapeDtypeStruct((N, 1), _f32),
    jax.ShapeDtypeStruct((1, 16), _f32),
]
_attn_outs_spec = [_spec_h4, _spec_rows, _spec_rows, _spec_whole((1, 16))]


def _pre_call(x, W, a_s, a_d):
    return pl.pallas_call(
        _pre_body,
        grid=(NBL,),
        in_specs=[pl.BlockSpec((BN, D), lambda i: (i, 0)),
                  _spec_whole((D, H)), _spec_whole((H, 1)), _spec_whole((H, 1))],
        out_specs=_attn_outs_spec,
        out_shape=_attn_outs_shape,
        scratch_shapes=[pltpu.SMEM((2,), _f32)],
    )(x, W, a_s.reshape(H, 1), a_d.reshape(H, 1))


def _mid_call(sums4, denc, b, g, be, W, a_s, a_d):
    return pl.pallas_call(
        _mid_body,
        grid=(NBL,),
        in_specs=[pl.BlockSpec((NQ, BN, DQ), lambda i: (0, i, 0)),
                  _spec_rows,
                  _spec_whole((1, H)), _spec_whole((1, H)), _spec_whole((1, H)),
                  _spec_whole((H, H)), _spec_whole((H, 1)), _spec_whole((H, 1))],
        out_specs=_attn_outs_spec,
        out_shape=_attn_outs_shape,
        scratch_shapes=[pltpu.SMEM((2,), _f32)],
    )(sums4, denc, b.reshape(1, H), g.reshape(1, H), be.reshape(1, H),
      W, a_s.reshape(H, 1), a_d.reshape(H, 1))


def _post_call(sums4, denc, b, g, be, batch, graph_attr, fc1W, fc1b, fc2W, fc2b):
    return pl.pallas_call(
        _post_body,
        grid=(NBL,),
        in_specs=[pl.BlockSpec((NQ, BN, DQ), lambda i: (0, i, 0)),
                  _spec_rows,
                  _spec_whole((1, H)), _spec_whole((1, H)), _spec_whole((1, H)),
                  pl.BlockSpec((1, 1, BN), lambda i: (i, 0, 0)),
                  _spec_whole((G, A)),
                  _spec_whole((H, H)), _spec_whole((A, H)),
                  _spec_whole((1, H)), _spec_whole((H, O)),
                  _spec_whole((1, O))],
        out_specs=_spec_whole((G, O)),
        out_shape=jax.ShapeDtypeStruct((G, O), _f32),
        scratch_shapes=[pltpu.VMEM((G, H), _f32), pltpu.VMEM((G, 1), _f32)],
    )(sums4, denc, b.reshape(1, H), g.reshape(1, H), be.reshape(1, H),
      batch.reshape(NBL, 1, BN), graph_attr.reshape(G, A),
      fc1W[:H], fc1W[H:], fc1b.reshape(1, H), fc2W, fc2b.reshape(1, O))


def kernel(x, edge_index, batch, graph_attr, W1, a1s, a1d, b1, g1, be1, W2, a2s, a2d, b2, g2, be2, W3, a3s, a3d, b3, g3, be3, fc1W, fc1b, fc2W, fc2b):
    loop = jnp.arange(N, dtype=jnp.int32)
    pad = jnp.zeros((ECP - EC,), jnp.int32)
    srcp = jnp.concatenate([edge_index[0].astype(jnp.int32), loop, pad])
    dstp = jnp.concatenate([edge_index[1].astype(jnp.int32), loop, pad])

    h4, asrc, adst, cv = _pre_call(x, W1, a1s, a1d)
    sums, den = _sc_edge(h4.reshape(NQ * N, DQ), asrc.reshape(N),
                         adst.reshape(N), srcp, dstp, cv.reshape(16))

    h4, asrc, adst, cv = _mid_call(sums.reshape(NQ, NP, DQ),
                                   den[0, :N].reshape(N, 1), b1, g1, be1,
                                   W2, a2s, a2d)
    sums, den = _sc_edge(h4.reshape(NQ * N, DQ), asrc.reshape(N),
                         adst.reshape(N), srcp, dstp, cv.reshape(16))

    h4, asrc, adst, cv = _mid_call(sums.reshape(NQ, NP, DQ),
                                   den[0, :N].reshape(N, 1), b2, g2, be2,
                                   W3, a3s, a3d)
    sums, den = _sc_edge(h4.reshape(NQ * N, DQ), asrc.reshape(N),
                         adst.reshape(N), srcp, dstp, cv.reshape(16))

    return _post_call(sums.reshape(NQ, NP, DQ), den[0, :N].reshape(N, 1),
                      b3, g3, be3, batch, graph_attr, fc1W, fc1b, fc2W, fc2b)


# speedup vs baseline: 1.0630x; 1.0035x over previous
import functools

import jax
import jax.numpy as jnp
from jax import lax
from jax.experimental import pallas as pl
from jax.experimental.pallas import tpu as pltpu
import jax.experimental.pallas.tpu_sc as plsc

N = 10000
E = 320000
D = 128
H = 256
G = 64
A = 8
O = 1

EC = E + N
NT = 16
NC = 2
EPT = 20736
ECP = NT * EPT
SB = 2304
NSB = EPT // SB
K = 96
NKB = SB // K
NP = 10240
RPT = NP // NT
NBUF = 4
NQ = 4
DQ = H // NQ

_f32 = jnp.float32


def _sc_edge_body(h4, asrc_h, adst_h, srcp, dstp, cvec_h,
                  sums_o, den_o, den_pub,
                  asrc_v, adst_v, den_v, ex_v, s_sb, d_sb, gidx_v, didx_v,
                  rows_v, part_v, acc_v, cvec_v, sem_g, sem_s,
                  out_acc):
    c = lax.axis_index("c")
    s = lax.axis_index("s")
    tb = s * EPT
    row0 = s * RPT

    zero16 = jnp.zeros((16,), _f32)

    def _zden(i, _):
        den_v[pl.ds(i * 16, 16)] = zero16
        return 0
    lax.fori_loop(0, NP // 16, _zden, 0)

    def _zzr(i, _):
        for j in range(DQ // 16):
            rows_v[0][i, pl.ds(j * 16, 16)] = zero16
        return 0
    lax.fori_loop(0, K, _zzr, 0)

    for p in range(RPT // K):
        pltpu.sync_copy(rows_v[0], out_acc.at[pl.ds(row0 + p * K, K)])
    _zr = RPT - (RPT // K) * K
    if _zr:
        pltpu.sync_copy(rows_v[0].at[pl.ds(0, _zr)],
                        out_acc.at[pl.ds(row0 + (RPT // K) * K, _zr)])

    pltpu.sync_copy(asrc_h, asrc_v)
    pltpu.sync_copy(adst_h, adst_v)
    pltpu.sync_copy(cvec_h, cvec_v)
    cval = cvec_v[...][0]

    _ns_p1 = jax.named_scope("sc_phase1")
    _ns_p1.__enter__()

    def _p1_sb(sb, _):
        base = tb + sb * SB
        pltpu.sync_copy(srcp.at[pl.ds(base, SB)], s_sb)
        pltpu.sync_copy(dstp.at[pl.ds(base, SB)], d_sb)

        def _p1(j, _):
            sv = s_sb[pl.ds(j * 16, 16)]
            dv = d_sb[pl.ds(j * 16, 16)]
            av = plsc.load_gather(asrc_v, [sv])
            bv = plsc.load_gather(adst_v, [dv])
            t = av + bv
            e = jnp.where(t >= 0.0, t, t * 0.2)
            ex = jnp.exp(e - cval)
            gid = base + j * 16 + lax.iota(jnp.int32, 16)
            ex = jnp.where(gid < EC, ex, 0.0)
            ex_v[pl.ds(sb * SB + j * 16, 16)] = ex
            plsc.addupdate_scatter(den_v, [dv], ex)
            return 0
        lax.fori_loop(0, SB // 16, _p1, 0)
        return 0
    lax.fori_loop(0, NSB, _p1_sb, 0)
    _ns_p1.__exit__(None, None, None)

    _ns_dm = jax.named_scope("sc_denmerge")
    _ns_dm.__enter__()
    pltpu.sync_copy(den_v, den_pub.at[c, s])
    plsc.subcore_barrier()

    pltpu.sync_copy(den_pub.at[c, 0, pl.ds(row0, RPT)], acc_v)
    for t in range(1, NT):
        pltpu.sync_copy(den_pub.at[c, t, pl.ds(row0, RPT)], part_v)

        def _acc(i, _):
            acc_v[pl.ds(i * 16, 16)] = (acc_v[pl.ds(i * 16, 16)]
                                        + part_v[pl.ds(i * 16, 16)])
            return 0
        lax.fori_loop(0, RPT // 16, _acc, 0)
    pltpu.sync_copy(acc_v, den_o.at[c, pl.ds(row0, RPT)])
    _ns_dm.__exit__(None, None, None)

    for p in range(2):
        _ns_p2 = jax.named_scope(f"sc_phase2_{p}")
        _ns_p2.__enter__()
        qbase = (2 * c + p) * N

        if p > 0:
            def _zzr2(i, _):
                for j in range(DQ // 16):
                    rows_v[0][i, pl.ds(j * 16, 16)] = zero16
                return 0
            lax.fori_loop(0, K, _zzr2, 0)
            for z in range(RPT // K):
                pltpu.sync_copy(rows_v[0], out_acc.at[pl.ds(row0 + z * K, K)])
            _zr2 = RPT - (RPT // K) * K
            if _zr2:
                pltpu.sync_copy(rows_v[0].at[pl.ds(0, _zr2)],
                                out_acc.at[pl.ds(row0 + (RPT // K) * K, _zr2)])
            plsc.subcore_barrier()

        def _p2_sb(sb, _, qbase=qbase):
            base = tb + sb * SB
            pltpu.sync_copy(srcp.at[pl.ds(base, SB)], s_sb)
            pltpu.sync_copy(dstp.at[pl.ds(base, SB)], d_sb)

            def _p2_grp(grp, _):
                gg = sb * (NKB // NBUF) + grp
                gds = []
                for t in range(NBUF):
                    @pl.when(gg > 0)
                    def _(t=t):
                        pltpu.make_async_copy(
                            rows_v[t], out_acc.at[didx_v[t]], sem_s[t]).wait()
                    koff = (grp * NBUF + t) * K
                    for j in range(K // 16):
                        gidx_v[t][pl.ds(j * 16, 16)] = (
                            s_sb[pl.ds(koff + j * 16, 16)] + qbase)
                        didx_v[t][pl.ds(j * 16, 16)] = (
                            d_sb[pl.ds(koff + j * 16, 16)])
                    gds.append(pltpu.async_copy(h4.at[gidx_v[t]], rows_v[t],
                                                sem_g[t]))
                for t in range(NBUF):
                    gds[t].wait()
                    exbase = sb * SB + (grp * NBUF + t) * K

                    def _row_grp(g, _, t=t, exbase=exbase):
                        bv = jnp.full((16,), exbase + g * 16, jnp.int32)
                        for l in range(16):
                            exb = plsc.load_gather(ex_v, [bv + l])
                            i = g * 16 + l
                            for j in range(DQ // 16):
                                rows_v[t][i, pl.ds(j * 16, 16)] = (
                                    rows_v[t][i, pl.ds(j * 16, 16)] * exb)
                        return 0
                    lax.fori_loop(0, K // 16, _row_grp, 0)
                    pltpu.async_copy(rows_v[t], out_acc.at[didx_v[t]],
                                     sem_s[t], add=True)
                return 0
            lax.fori_loop(0, NKB // NBUF, _p2_grp, 0)
            return 0
        lax.fori_loop(0, NSB, _p2_sb, 0)

        for t in range(NBUF):
            pltpu.make_async_copy(rows_v[t], out_acc.at[didx_v[t]],
                                  sem_s[t]).wait()

        plsc.subcore_barrier()

        pltpu.sync_copy(out_acc.at[pl.ds(row0, RPT)],
                        sums_o.at[pl.ds((2 * c + p) * NP + row0, RPT)])
        _ns_p2.__exit__(None, None, None)


@functools.partial(jax.jit, static_argnames=())
def _sc_edge(h4, asrc, adst, srcp, dstp, cvec):
    mesh = plsc.VectorSubcoreMesh(core_axis_name="c", subcore_axis_name="s")
    return pl.kernel(
        _sc_edge_body,
        out_type=[
            jax.ShapeDtypeStruct((NQ * NP, DQ), _f32),
            jax.ShapeDtypeStruct((NC, NP), _f32),
            jax.ShapeDtypeStruct((NC, NT, NP), _f32),
        ],
        mesh=mesh,
        compiler_params=pltpu.CompilerParams(needs_layout_passes=False,
                                             use_tc_tiling_on_sc=False),
        scratch_types=[
            pltpu.VMEM((N,), _f32),
            pltpu.VMEM((N,), _f32),
            pltpu.VMEM((NP,), _f32),
            pltpu.VMEM((EPT,), _f32),
            pltpu.VMEM((SB,), jnp.int32),
            pltpu.VMEM((SB,), jnp.int32),
            [pltpu.VMEM((K,), jnp.int32) for _ in range(NBUF)],
            [pltpu.VMEM((K,), jnp.int32) for _ in range(NBUF)],
            [pltpu.VMEM((K, DQ), _f32) for _ in range(NBUF)],
            pltpu.VMEM((RPT,), _f32),
            pltpu.VMEM((RPT,), _f32),
            pltpu.VMEM((16,), _f32),
            [pltpu.SemaphoreType.DMA for _ in range(NBUF)],
            [pltpu.SemaphoreType.DMA for _ in range(NBUF)],
            pltpu.VMEM_SHARED((NP, DQ), _f32),
        ],
    )(h4, asrc, adst, srcp, dstp, cvec)[:2]


BN = 2000
NBL = N // BN


def _leaky(t):
    return jnp.where(t >= 0.0, t, t * 0.2)


def _attn_block(i, hn, asn_ref, adn_ref, asrc_ref, adst_ref, c_ref, mx_ref):
    asrc = jnp.dot(hn, asn_ref[...], preferred_element_type=_f32, precision=lax.Precision.HIGHEST)
    adst = jnp.dot(hn, adn_ref[...], preferred_element_type=_f32, precision=lax.Precision.HIGHEST)
    asrc_ref[...] = asrc
    adst_ref[...] = adst
    bs = jnp.max(asrc)
    bd = jnp.max(adst)

    @pl.when(i == 0)
    def _():
        mx_ref[0] = bs
        mx_ref[1] = bd

    @pl.when(i > 0)
    def _():
        mx_ref[0] = jnp.maximum(mx_ref[0], bs)
        mx_ref[1] = jnp.maximum(mx_ref[1], bd)

    @pl.when(i == NBL - 1)
    def _():
        c_ref[...] = jnp.full((1, 16), _leaky(mx_ref[0] + mx_ref[1]), _f32)


def _pre_body(x_ref, w_ref, asn_ref, adn_ref,
              h4_ref, asrc_ref, adst_ref, c_ref, mx_ref):
    i = pl.program_id(0)
    hn = jnp.dot(x_ref[...], w_ref[...], preferred_element_type=_f32, precision=lax.Precision.HIGHEST)
    for q in range(NQ):
        h4_ref[q] = hn[:, q * DQ:(q + 1) * DQ]
    _attn_block(i, hn, asn_ref, adn_ref, asrc_ref, adst_ref, c_ref, mx_ref)


def _norm_block(sums_ref, denc, b, g, be):
    u = jnp.concatenate([sums_ref[q] for q in range(NQ)], axis=1)
    v = u / (denc + 1e-16) + b
    mu = jnp.mean(v, axis=1, keepdims=True)
    var = jnp.mean((v - mu) ** 2, axis=1, keepdims=True)
    ln = (v - mu) / jnp.sqrt(var + 1e-5) * g + be
    return jnp.maximum(ln, 0.0)


def _mid_body(sums_ref, denc_ref, b_ref, g_ref, be_ref, w_ref, asn_ref, adn_ref,
              h4_ref, asrc_ref, adst_ref, c_ref, mx_ref):
    i = pl.program_id(0)
    r = _norm_block(sums_ref, denc_ref[...], b_ref[...], g_ref[...], be_ref[...])
    hn = jnp.dot(r, w_ref[...], preferred_element_type=_f32, precision=lax.Precision.HIGHEST)
    for q in range(NQ):
        h4_ref[q] = hn[:, q * DQ:(q + 1) * DQ]
    _attn_block(i, hn, asn_ref, adn_ref, asrc_ref, adst_ref, c_ref, mx_ref)


def _post_body(sums_ref, denc_ref, b_ref, g_ref, be_ref, batch_ref, ga_ref,
               f1a_ref, f1b_ref, f1bias_ref, f2w_ref, f2b_ref, out_ref,
               pool_ref, cnt_ref):
    i = pl.program_id(0)
    h = _norm_block(sums_ref, denc_ref[...], b_ref[...], g_ref[...],
                    be_ref[...])
    iota_g = lax.broadcasted_iota(jnp.int32, (G, BN), 0)
    onehot = (batch_ref[0] == iota_g).astype(_f32)
    cnt = jnp.sum(onehot, axis=1, keepdims=True)
    pooled = jnp.dot(onehot, h, preferred_element_type=_f32, precision=lax.Precision.HIGHEST)

    @pl.when(i == 0)
    def _():
        pool_ref[...] = pooled
        cnt_ref[...] = cnt

    @pl.when(i > 0)
    def _():
        pool_ref[...] = pool_ref[...] + pooled
        cnt_ref[...] = cnt_ref[...] + cnt

    @pl.when(i == NBL - 1)
    def _():
        pooled_m = pool_ref[...] / jnp.maximum(cnt_ref[...], 1.0)
        _hp = functools.partial(jnp.dot, preferred_element_type=_f32,
                                precision=lax.Precision.HIGHEST)
        z = (_hp(pooled_m, f1a_ref[...]) + _hp(ga_ref[...], f1b_ref[...])
             + f1bias_ref[...])
        z = jnp.maximum(z, 0.0)
        out_ref[...] = _hp(z, f2w_ref[...]) + f2b_ref[...]


_spec_whole = lambda shape: pl.BlockSpec(shape, lambda i: (0,) * len(shape))
_spec_rows = pl.BlockSpec((BN, 1), lambda i: (i, 0))
_spec_h4 = pl.BlockSpec((NQ, BN, DQ), lambda i: (0, i, 0))

_attn_outs_shape = [
    jax.ShapeDtypeStruct((NQ, N, DQ), _f32),
    jax.ShapeDtypeStruct((N, 1), _f32),
    jax.ShapeDtypeStruct((N, 1), _f32),
    jax.ShapeDtypeStruct((1, 16), _f32),
]
_attn_outs_spec = [_spec_h4, _spec_rows, _spec_rows, _spec_whole((1, 16))]


def _pre_call(x, W, a_s, a_d):
    return pl.pallas_call(
        _pre_body,
        grid=(NBL,),
        in_specs=[pl.BlockSpec((BN, D), lambda i: (i, 0)),
                  _spec_whole((D, H)), _spec_whole((H, 1)), _spec_whole((H, 1))],
        out_specs=_attn_outs_spec,
        out_shape=_attn_outs_shape,
        scratch_shapes=[pltpu.SMEM((2,), _f32)],
    )(x, W, a_s.reshape(H, 1), a_d.reshape(H, 1))


def _mid_call(sums4, denc, b, g, be, W, a_s, a_d):
    return pl.pallas_call(
        _mid_body,
        grid=(NBL,),
        in_specs=[pl.BlockSpec((NQ, BN, DQ), lambda i: (0, i, 0)),
                  _spec_rows,
                  _spec_whole((1, H)), _spec_whole((1, H)), _spec_whole((1, H)),
                  _spec_whole((H, H)), _spec_whole((H, 1)), _spec_whole((H, 1))],
        out_specs=_attn_outs_spec,
        out_shape=_attn_outs_shape,
        scratch_shapes=[pltpu.SMEM((2,), _f32)],
    )(sums4, denc, b.reshape(1, H), g.reshape(1, H), be.reshape(1, H),
      W, a_s.reshape(H, 1), a_d.reshape(H, 1))


def _post_call(sums4, denc, b, g, be, batch, graph_attr, fc1W, fc1b, fc2W, fc2b):
    return pl.pallas_call(
        _post_body,
        grid=(NBL,),
        in_specs=[pl.BlockSpec((NQ, BN, DQ), lambda i: (0, i, 0)),
                  _spec_rows,
                  _spec_whole((1, H)), _spec_whole((1, H)), _spec_whole((1, H)),
                  pl.BlockSpec((1, 1, BN), lambda i: (i, 0, 0)),
                  _spec_whole((G, A)),
                  _spec_whole((H, H)), _spec_whole((A, H)),
                  _spec_whole((1, H)), _spec_whole((H, O)),
                  _spec_whole((1, O))],
        out_specs=_spec_whole((G, O)),
        out_shape=jax.ShapeDtypeStruct((G, O), _f32),
        scratch_shapes=[pltpu.VMEM((G, H), _f32), pltpu.VMEM((G, 1), _f32)],
    )(sums4, denc, b.reshape(1, H), g.reshape(1, H), be.reshape(1, H),
      batch.reshape(NBL, 1, BN), graph_attr.reshape(G, A),
      fc1W[:H], fc1W[H:], fc1b.reshape(1, H), fc2W, fc2b.reshape(1, O))


def kernel(x, edge_index, batch, graph_attr, W1, a1s, a1d, b1, g1, be1, W2, a2s, a2d, b2, g2, be2, W3, a3s, a3d, b3, g3, be3, fc1W, fc1b, fc2W, fc2b):
    loop = jnp.arange(N, dtype=jnp.int32)
    pad = jnp.zeros((ECP - EC,), jnp.int32)
    srcp = jnp.concatenate([edge_index[0].astype(jnp.int32), loop, pad])
    dstp = jnp.concatenate([edge_index[1].astype(jnp.int32), loop, pad])

    h4, asrc, adst, cv = _pre_call(x, W1, a1s, a1d)
    sums, den = _sc_edge(h4.reshape(NQ * N, DQ), asrc.reshape(N),
                         adst.reshape(N), srcp, dstp, cv.reshape(16))

    h4, asrc, adst, cv = _mid_call(sums.reshape(NQ, NP, DQ),
                                   den[0, :N].reshape(N, 1), b1, g1, be1,
                                   W2, a2s, a2d)
    sums, den = _sc_edge(h4.reshape(NQ * N, DQ), asrc.reshape(N),
                         adst.reshape(N), srcp, dstp, cv.reshape(16))

    h4, asrc, adst, cv = _mid_call(sums.reshape(NQ, NP, DQ),
                                   den[0, :N].reshape(N, 1), b2, g2, be2,
                                   W3, a3s, a3d)
    sums, den = _sc_edge(h4.reshape(NQ * N, DQ), asrc.reshape(N),
                         adst.reshape(N), srcp, dstp, cv.reshape(16))

    return _post_call(sums.reshape(NQ, NP, DQ), den[0, :N].reshape(N, 1),
                      b3, g3, be3, batch, graph_attr, fc1W, fc1b, fc2W, fc2b)
